# baseline (speedup 1.0000x reference)
import jax
import jax.numpy as jnp
from jax.experimental import pallas as pl
from jax.experimental.pallas import tpu as pltpu

S = 64.0
M = 0.4

_BLOCK_C = 1024


def _body(lab_ref, cos_ref, out_ref):
    i = pl.program_id(0)
    lab = lab_ref[:, 0]
    rows = i * _BLOCK_C + jax.lax.broadcasted_iota(
        jnp.int32, (_BLOCK_C, lab_ref.shape[0]), 0)
    mask = (rows == lab[None, :]).astype(jnp.float32)
    out_ref[...] = S * cos_ref[...] - (S * M) * mask


def kernel(cosine, label):
    B, C = cosine.shape
    cos_t = cosine.T
    out_t = pl.pallas_call(
        _body,
        grid=(pl.cdiv(C, _BLOCK_C),),
        in_specs=[
            pl.BlockSpec((B, 1), lambda i: (0, 0)),
            pl.BlockSpec((_BLOCK_C, B), lambda i: (i, 0)),
        ],
        out_specs=pl.BlockSpec((_BLOCK_C, B), lambda i: (i, 0)),
        out_shape=jax.ShapeDtypeStruct((C, B), cosine.dtype),
        compiler_params=pltpu.CompilerParams(
            dimension_semantics=("parallel",),
            vmem_limit_bytes=96 * 1024 * 1024,
        ),
    )(label.reshape(B, 1), cos_t)
    return out_t.T


# speedup vs baseline: 1.0638x; 1.0638x over previous
import jax
import jax.numpy as jnp
from jax.experimental import pallas as pl
from jax.experimental.pallas import tpu as pltpu

S = 64.0
M = 0.4

_BLOCK_C = 2048


def _body(lab_ref, cos_ref, out_ref):
    i = pl.program_id(0)
    lab = lab_ref[:, 0]
    rows = i * _BLOCK_C + jax.lax.broadcasted_iota(
        jnp.int32, (_BLOCK_C, lab_ref.shape[0]), 0)
    mask = (rows == lab[None, :]).astype(jnp.float32)
    out_ref[...] = S * cos_ref[...] - (S * M) * mask


def kernel(cosine, label):
    B, C = cosine.shape
    cos_t = cosine.T
    out_t = pl.pallas_call(
        _body,
        grid=(pl.cdiv(C, _BLOCK_C),),
        in_specs=[
            pl.BlockSpec((B, 1), lambda i: (0, 0)),
            pl.BlockSpec((_BLOCK_C, B), lambda i: (i, 0)),
        ],
        out_specs=pl.BlockSpec((_BLOCK_C, B), lambda i: (i, 0)),
        out_shape=jax.ShapeDtypeStruct((C, B), cosine.dtype),
        compiler_params=pltpu.CompilerParams(
            dimension_semantics=("parallel",),
            vmem_limit_bytes=96 * 1024 * 1024,
        ),
    )(label.reshape(B, 1), cos_t)
    return out_t.T


# speedup vs baseline: 1.0767x; 1.0121x over previous
import jax
import jax.numpy as jnp
from jax.experimental import pallas as pl
from jax.experimental.pallas import tpu as pltpu

S = 64.0
M = 0.4

_BLOCK_C = 2000


def _body(lab_ref, cos_ref, out_ref):
    i = pl.program_id(0)
    lab = lab_ref[:, 0]
    rows = i * _BLOCK_C + jax.lax.broadcasted_iota(
        jnp.int32, (_BLOCK_C, lab_ref.shape[0]), 0)
    mask = (rows == lab[None, :]).astype(jnp.float32)
    out_ref[...] = S * cos_ref[...] - (S * M) * mask


def kernel(cosine, label):
    B, C = cosine.shape
    cos_t = cosine.T
    out_t = pl.pallas_call(
        _body,
        grid=(pl.cdiv(C, _BLOCK_C),),
        in_specs=[
            pl.BlockSpec((B, 1), lambda i: (0, 0)),
            pl.BlockSpec((_BLOCK_C, B), lambda i: (i, 0)),
        ],
        out_specs=pl.BlockSpec((_BLOCK_C, B), lambda i: (i, 0)),
        out_shape=jax.ShapeDtypeStruct((C, B), cosine.dtype),
        compiler_params=pltpu.CompilerParams(
            dimension_semantics=("parallel",),
            vmem_limit_bytes=96 * 1024 * 1024,
        ),
    )(label.reshape(B, 1), cos_t)
    return out_t.T
